# Initial kernel scaffold; baseline (speedup 1.0000x reference)
import functools

import jax
import jax.numpy as jnp
from jax import lax
from jax.experimental import pallas as pl
from jax.experimental.pallas import tpu as pltpu
from jax.experimental.pallas import tpu_sc as plsc

N = 10000
NP = 10240
E_RAW = 320000
E_TOT = E_RAW + N
C = 512
NCHUNK16 = 41
E_PAD = NCHUNK16 * 16 * C
HEADS, HID, OUT = 8, 8, 64
NHEAD, NLAYERS, DFF = 4, 3, 128
FRAMES = 5
L = 16
BL = 2048

f32 = jnp.float32


def _tc1_body(x_ref, w_ref, as_ref, ad_ref, h_ref, als_ref, ald_ref):
    xb = x_ref[0]
    h = lax.dot_general(w_ref[...], xb, (((1,), (1,)), ((), ())),
                        preferred_element_type=f32)
    h_ref[0] = h
    als_ref[0] = jnp.dot(as_ref[...], h, preferred_element_type=f32)
    ald_ref[0] = jnp.dot(ad_ref[...], h, preferred_element_type=f32)


def _tc1(xp, W1T, AsT, AdT):
    grid = (FRAMES, NP // BL)
    return pl.pallas_call(
        _tc1_body,
        grid=grid,
        in_specs=[
            pl.BlockSpec((1, BL, 128), lambda f, j: (f, j, 0)),
            pl.BlockSpec((OUT, 128), lambda f, j: (0, 0)),
            pl.BlockSpec((HEADS, OUT), lambda f, j: (0, 0)),
            pl.BlockSpec((HEADS, OUT), lambda f, j: (0, 0)),
        ],
        out_specs=[
            pl.BlockSpec((1, OUT, BL), lambda f, j: (f, 0, j)),
            pl.BlockSpec((1, HEADS, BL), lambda f, j: (f, 0, j)),
            pl.BlockSpec((1, HEADS, BL), lambda f, j: (f, 0, j)),
        ],
        out_shape=[
            jax.ShapeDtypeStruct((FRAMES, OUT, NP), f32),
            jax.ShapeDtypeStruct((FRAMES, HEADS, NP), f32),
            jax.ShapeDtypeStruct((FRAMES, HEADS, NP), f32),
        ],
    )(xp, W1T, AsT, AdT)


def _tc2_body(o_ref, b1_ref, w2_ref, as2_ref, ad2_ref, h2_ref, als_ref, ald_ref):
    h = o_ref[...] + b1_ref[...]
    h = jnp.where(h > 0, h, jnp.expm1(h))
    h2 = jnp.dot(w2_ref[...], h, preferred_element_type=f32)
    h2_ref[...] = h2
    als_ref[...] = jnp.dot(as2_ref[...], h2, preferred_element_type=f32)
    ald_ref[...] = jnp.dot(ad2_ref[...], h2, preferred_element_type=f32)


def _tc2(out1T, b1c, W2T, as2, ad2):
    return pl.pallas_call(
        _tc2_body,
        grid=(NP // BL,),
        in_specs=[
            pl.BlockSpec((OUT, BL), lambda j: (0, j)),
            pl.BlockSpec((OUT, 1), lambda j: (0, 0)),
            pl.BlockSpec((OUT, OUT), lambda j: (0, 0)),
            pl.BlockSpec((1, OUT), lambda j: (0, 0)),
            pl.BlockSpec((1, OUT), lambda j: (0, 0)),
        ],
        out_specs=[
            pl.BlockSpec((OUT, BL), lambda j: (0, j)),
            pl.BlockSpec((1, BL), lambda j: (0, j)),
            pl.BlockSpec((1, BL), lambda j: (0, j)),
        ],
        out_shape=[
            jax.ShapeDtypeStruct((OUT, NP), f32),
            jax.ShapeDtypeStruct((1, NP), f32),
            jax.ShapeDtypeStruct((1, NP), f32),
        ],
    )(out1T, b1c, W2T, as2, ad2)


def _ln_in(h, g, b):
    m = jnp.mean(h, axis=1, keepdims=True)
    v = jnp.mean((h - m) ** 2, axis=1, keepdims=True)
    return (h - m) * lax.rsqrt(v + 1e-5) * g + b


def _tc3_body(h_ref, Wq_ref, bq_ref, Wk_ref, bk_ref, Wv_ref, bv_ref,
              Wo_ref, bo_ref, g1_ref, be1_ref, g2_ref, be2_ref,
              Wf1_ref, bf1_ref, Wf2_ref, bf2_ref,
              Wh1_ref, bh1_ref, Wh2_ref, bh2_ref, out_ref):
    h = h_ref[...]
    hd = OUT // NHEAD
    for layer in range(NLAYERS):
        q = jnp.dot(h, Wq_ref[layer], preferred_element_type=f32) + bq_ref[layer]
        k = jnp.dot(h, Wk_ref[layer], preferred_element_type=f32) + bk_ref[layer]
        v = jnp.dot(h, Wv_ref[layer], preferred_element_type=f32) + bv_ref[layer]
        outs = []
        for hh in range(NHEAD):
            sl = slice(hh * hd, (hh + 1) * hd)
            qh, kh, vh = q[:, sl], k[:, sl], v[:, sl]
            att = lax.dot_general(qh, kh, (((1,), (1,)), ((), ())),
                                  preferred_element_type=f32) * (1.0 / (hd ** 0.5))
            att = att - jnp.max(att, axis=1, keepdims=True)
            att = jnp.exp(att)
            att = att / jnp.sum(att, axis=1, keepdims=True)
            outs.append(lax.dot_general(att, vh, (((1,), (0,)), ((), ())),
                                        preferred_element_type=f32))
        o = jnp.concatenate(outs, axis=1)
        o = jnp.dot(o, Wo_ref[layer], preferred_element_type=f32) + bo_ref[layer]
        h = _ln_in(h + o, g1_ref[layer], be1_ref[layer])
        f1 = jnp.maximum(jnp.dot(h, Wf1_ref[layer], preferred_element_type=f32)
                         + bf1_ref[layer], 0.0)
        fo = jnp.dot(f1, Wf2_ref[layer], preferred_element_type=f32) + bf2_ref[layer]
        h = _ln_in(h + fo, g2_ref[layer], be2_ref[layer])
    last = h[FRAMES - 1:FRAMES, :]
    r = jnp.maximum(jnp.dot(last, Wh1_ref[...], preferred_element_type=f32)
                    + bh1_ref[...], 0.0)
    out_ref[...] = jnp.dot(r, Wh2_ref[...], preferred_element_type=f32) + bh2_ref[...]


def _tc3(hseq, Wq, bq, Wk, bk, Wv, bv, Wo, bo, g1, be1, g2, be2,
         Wf1, bf1, Wf2, bf2, Wh1, bh1, Wh2, bh2):
    return pl.pallas_call(
        _tc3_body,
        out_shape=jax.ShapeDtypeStruct((1, 2), f32),
    )(hseq, Wq, bq, Wk, bk, Wv, bv, Wo, bo, g1, be1, g2, be2,
      Wf1, bf1, Wf2, bf2, Wh1, bh1, Wh2, bh2)


_MESH = plsc.VectorSubcoreMesh(core_axis_name="c", subcore_axis_name="s")


def _zero_range(ref, n):
    z = jnp.zeros((L,), f32)

    @pl.loop(0, n // L)
    def _(i):
        ref[pl.ds(i * L, L)] = z


def _edge_stream(src_h, dst_h, srcb, dstb, sems, semd, base, nch, process):
    pltpu.async_copy(src_h.at[pl.ds(base, C)], srcb.at[0], sems.at[0])
    pltpu.async_copy(dst_h.at[pl.ds(base, C)], dstb.at[0], semd.at[0])

    @pl.loop(0, nch, step=2)
    def _(ci):
        off0 = base + ci * C
        pltpu.make_async_copy(src_h.at[pl.ds(off0, C)], srcb.at[0], sems.at[0]).wait()
        pltpu.make_async_copy(dst_h.at[pl.ds(off0, C)], dstb.at[0], semd.at[0]).wait()

        @pl.when(ci + 1 < nch)
        def _():
            pltpu.async_copy(src_h.at[pl.ds(off0 + C, C)], srcb.at[1], sems.at[1])
            pltpu.async_copy(dst_h.at[pl.ds(off0 + C, C)], dstb.at[1], semd.at[1])

        process(off0, 0)

        @pl.when(ci + 1 < nch)
        def _():
            pltpu.make_async_copy(src_h.at[pl.ds(off0 + C, C)], srcb.at[1], sems.at[1]).wait()
            pltpu.make_async_copy(dst_h.at[pl.ds(off0 + C, C)], dstb.at[1], semd.at[1]).wait()

            @pl.when(ci + 2 < nch)
            def _():
                pltpu.async_copy(src_h.at[pl.ds(off0 + 2 * C, C)], srcb.at[0], sems.at[0])
                pltpu.async_copy(dst_h.at[pl.ds(off0 + 2 * C, C)], dstb.at[0], semd.at[0])

            process(off0 + C, 1)


_SC_SCRATCH = [
    pltpu.VMEM((NP,), f32),
    pltpu.VMEM((NP,), f32),
    pltpu.VMEM((NP,), f32),
    pltpu.VMEM((NP,), f32),
    pltpu.VMEM((NP,), f32),
    pltpu.VMEM((NP,), f32),
    pltpu.VMEM((NP,), f32),
    pltpu.VMEM((NP // 4,), f32),
    pltpu.VMEM((2, C), jnp.int32),
    pltpu.VMEM((2, C), jnp.int32),
    pltpu.VMEM_SHARED((16, NP), f32),
    pltpu.VMEM_SHARED((4, NP), f32),
    pltpu.SemaphoreType.DMA((2,)),
    pltpu.SemaphoreType.DMA((2,)),
]


@functools.partial(
    pl.kernel,
    out_type=jax.ShapeDtypeStruct((OUT, NP), f32),
    mesh=_MESH,
    scratch_types=_SC_SCRATCH,
)
def _sc_l1(src_h, dst_h, alsT_h, aldT_h, h1T_h, out_h,
           als_t, ald_t, den_t, hc0, hc1, o0, o1, tmpc, srcb, dstb,
           shpart, shcomb, sems, semd):
    c = lax.axis_index("c")
    s = lax.axis_index("s")
    hloc = s // 4
    ghead = 4 * c + hloc
    col0 = 32 * c + 2 * s

    pltpu.sync_copy(alsT_h.at[ghead], als_t)
    pltpu.sync_copy(aldT_h.at[ghead], ald_t)
    _zero_range(den_t, NP)
    _zero_range(o0, NP)
    _zero_range(o1, NP)

    qa = s % 4
    EQ = E_PAD // 4

    def _procA(off, b):
        for j in range(C // L):
            s16 = srcb[b, pl.ds(j * L, L)]
            d16 = dstb[b, pl.ds(j * L, L)]
            a = plsc.load_gather(als_t, [s16]) + plsc.load_gather(ald_t, [d16])
            a = jnp.where(a > 0, a, 0.2 * a)
            ex = jnp.exp(a)
            eid = off + j * L + lax.iota(jnp.int32, L)
            plsc.addupdate_scatter(den_t, [d16], ex, mask=eid < E_TOT)

    _edge_stream(src_h, dst_h, srcb, dstb, sems, semd, qa * EQ, EQ // C, _procA)

    pltpu.sync_copy(den_t, shpart.at[s])
    plsc.subcore_barrier()
    SL1 = NP // 4
    offn = qa * SL1
    pltpu.sync_copy(shpart.at[4 * hloc, pl.ds(offn, SL1)], hc0.at[pl.ds(0, SL1)])
    for k2 in range(1, 4):
        pltpu.sync_copy(shpart.at[4 * hloc + k2, pl.ds(offn, SL1)],
                        tmpc.at[pl.ds(0, SL1)])

        @pl.loop(0, SL1 // L)
        def _(i):
            hc0[pl.ds(i * L, L)] = hc0[pl.ds(i * L, L)] + tmpc[pl.ds(i * L, L)]

    pltpu.sync_copy(hc0.at[pl.ds(0, SL1)], shcomb.at[hloc, pl.ds(offn, SL1)])
    plsc.subcore_barrier()
    pltpu.sync_copy(shcomb.at[hloc], den_t)

    pltpu.sync_copy(h1T_h.at[col0], hc0)
    pltpu.sync_copy(h1T_h.at[col0 + 1], hc1)

    def _procB(off, b):
        for j in range(C // L):
            s16 = srcb[b, pl.ds(j * L, L)]
            d16 = dstb[b, pl.ds(j * L, L)]
            a = plsc.load_gather(als_t, [s16]) + plsc.load_gather(ald_t, [d16])
            a = jnp.where(a > 0, a, 0.2 * a)
            ex = jnp.exp(a)
            dn = plsc.load_gather(den_t, [d16])
            coef = ex / (dn + 1e-16)
            eid = off + j * L + lax.iota(jnp.int32, L)
            valid = eid < E_TOT
            g0 = plsc.load_gather(hc0, [s16])
            g1 = plsc.load_gather(hc1, [s16])
            plsc.addupdate_scatter(o0, [d16], coef * g0, mask=valid)
            plsc.addupdate_scatter(o1, [d16], coef * g1, mask=valid)

    _edge_stream(src_h, dst_h, srcb, dstb, sems, semd, 0, E_PAD // C, _procB)

    pltpu.sync_copy(o0, out_h.at[col0])
    pltpu.sync_copy(o1, out_h.at[col0 + 1])


@functools.partial(
    pl.kernel,
    out_type=jax.ShapeDtypeStruct((OUT, L), f32),
    mesh=_MESH,
    scratch_types=_SC_SCRATCH,
)
def _sc_l2(src_h, dst_h, als2_h, ald2_h, h2T_h, out_h,
           als_t, ald_t, den_t, hc0, hc1, o0, o1, tmpc, srcb, dstb,
           shpart, shcomb, sems, semd):
    c = lax.axis_index("c")
    s = lax.axis_index("s")
    col0 = 32 * c + 2 * s

    pltpu.sync_copy(als2_h, als_t)
    pltpu.sync_copy(ald2_h, ald_t)
    _zero_range(den_t, NP)

    ES = E_PAD // 16

    def _procA(off, b):
        for j in range(C // L):
            s16 = srcb[b, pl.ds(j * L, L)]
            d16 = dstb[b, pl.ds(j * L, L)]
            a = plsc.load_gather(als_t, [s16]) + plsc.load_gather(ald_t, [d16])
            a = jnp.where(a > 0, a, 0.2 * a)
            ex = jnp.exp(a)
            eid = off + j * L + lax.iota(jnp.int32, L)
            plsc.addupdate_scatter(den_t, [d16], ex, mask=eid < E_TOT)

    _edge_stream(src_h, dst_h, srcb, dstb, sems, semd, s * ES, ES // C, _procA)

    pltpu.sync_copy(den_t, shpart.at[s])
    plsc.subcore_barrier()
    SL2 = NP // 16
    offn = s * SL2
    pltpu.sync_copy(shpart.at[0, pl.ds(offn, SL2)], hc0.at[pl.ds(0, SL2)])
    for k2 in range(1, 16):
        pltpu.sync_copy(shpart.at[k2, pl.ds(offn, SL2)], tmpc.at[pl.ds(0, SL2)])

        @pl.loop(0, SL2 // L)
        def _(i):
            hc0[pl.ds(i * L, L)] = hc0[pl.ds(i * L, L)] + tmpc[pl.ds(i * L, L)]

    pltpu.sync_copy(hc0.at[pl.ds(0, SL2)], shcomb.at[0, pl.ds(offn, SL2)])
    plsc.subcore_barrier()
    pltpu.sync_copy(shcomb.at[0], den_t)

    pltpu.sync_copy(h2T_h.at[col0], hc0)
    pltpu.sync_copy(h2T_h.at[col0 + 1], hc1)

    def _procB(off, b):
        acc0 = o0[pl.ds(0, L)]
        acc1 = o1[pl.ds(0, L)]
        for j in range(C // L):
            s16 = srcb[b, pl.ds(j * L, L)]
            d16 = dstb[b, pl.ds(j * L, L)]
            a = plsc.load_gather(als_t, [s16]) + plsc.load_gather(ald_t, [d16])
            a = jnp.where(a > 0, a, 0.2 * a)
            ex = jnp.exp(a)
            dn = plsc.load_gather(den_t, [d16])
            eid = off + j * L + lax.iota(jnp.int32, L)
            coef = jnp.where(eid < E_TOT, ex / (dn + 1e-16), 0.0)
            acc0 = acc0 + coef * plsc.load_gather(hc0, [s16])
            acc1 = acc1 + coef * plsc.load_gather(hc1, [s16])
        o0[pl.ds(0, L)] = acc0
        o1[pl.ds(0, L)] = acc1

    z = jnp.zeros((L,), f32)
    o0[pl.ds(0, L)] = z
    o1[pl.ds(0, L)] = z
    _edge_stream(src_h, dst_h, srcb, dstb, sems, semd, 0, E_PAD // C, _procB)

    pltpu.sync_copy(o0.at[pl.ds(0, L)], out_h.at[col0])
    pltpu.sync_copy(o1.at[pl.ds(0, L)], out_h.at[col0 + 1])


def kernel(x, edge_index, W1, att_src1, att_dst1, b1, W2, att_src2, att_dst2,
           b2, Wq, bq, Wk, bk, Wv, bv, Wo, bo, ln1_g, ln1_b, ln2_g, ln2_b,
           Wff1, bff1, Wff2, bff2, Whead1, bhead1, Whead2, bhead2):
    xp = jnp.pad(x, ((0, 0), (0, NP - N), (0, 0)))
    W1T = W1.T
    eye8 = jnp.eye(HEADS, dtype=f32)
    AsT = (eye8[:, :, None] * att_src1[None, :, :]).reshape(HEADS, OUT)
    AdT = (eye8[:, :, None] * att_dst1[None, :, :]).reshape(HEADS, OUT)
    h1T, alsT, aldT = _tc1(xp, W1T, AsT, AdT)

    loop = jnp.arange(N, dtype=jnp.int32)
    loops = jnp.broadcast_to(loop, (FRAMES, N))
    srcs = jnp.pad(jnp.concatenate([edge_index[:, 0, :], loops], axis=1),
                   ((0, 0), (0, E_PAD - E_TOT)))
    dsts = jnp.pad(jnp.concatenate([edge_index[:, 1, :], loops], axis=1),
                   ((0, 0), (0, E_PAD - E_TOT)))

    W2T = W2.T
    b1c = b1.reshape(OUT, 1)
    embs = []
    for f in range(FRAMES):
        out1T = _sc_l1(srcs[f], dsts[f], alsT[f], aldT[f], h1T[f])
        h2T, als2, ald2 = _tc2(out1T, b1c, W2T, att_src2, att_dst2)
        sc2 = _sc_l2(srcs[f], dsts[f], als2.reshape(NP), ald2.reshape(NP), h2T)
        embs.append(sc2.sum(axis=1) * (1.0 / N) + b2)
    hseq = jnp.stack(embs)

    return _tc3(hseq, Wq, bq, Wk, bk, Wv, bv, Wo, bo,
                ln1_g, ln1_b, ln2_g, ln2_b, Wff1, bff1, Wff2, bff2,
                Whead1, bhead1, Whead2, bhead2)


# speedup vs baseline: 37.1261x; 37.1261x over previous
import functools

import jax
import jax.numpy as jnp
from jax import lax
from jax.experimental import pallas as pl
from jax.experimental.pallas import tpu as pltpu
from jax.experimental.pallas import tpu_sc as plsc

N = 10000
NP = 10240
E_RAW = 320000
E_TOT = E_RAW + N
C = 512
NCHUNK16 = 41
E_PAD = NCHUNK16 * 16 * C
HEADS, HID, OUT = 8, 8, 64
NHEAD, NLAYERS, DFF = 4, 3, 128
FRAMES = 5
L = 16
BL = 2048

f32 = jnp.float32


def _tc1_body(x_ref, w_ref, as_ref, ad_ref, h_ref, als_ref, ald_ref):
    xb = x_ref[0]
    h = lax.dot_general(w_ref[...], xb, (((1,), (1,)), ((), ())),
                        preferred_element_type=f32)
    h_ref[0] = h
    als_ref[0] = jnp.dot(as_ref[...], h, preferred_element_type=f32)
    ald_ref[0] = jnp.dot(ad_ref[...], h, preferred_element_type=f32)


def _tc1(xp, W1T, AsT, AdT):
    grid = (FRAMES, NP // BL)
    return pl.pallas_call(
        _tc1_body,
        grid=grid,
        in_specs=[
            pl.BlockSpec((1, BL, 128), lambda f, j: (f, j, 0)),
            pl.BlockSpec((OUT, 128), lambda f, j: (0, 0)),
            pl.BlockSpec((HEADS, OUT), lambda f, j: (0, 0)),
            pl.BlockSpec((HEADS, OUT), lambda f, j: (0, 0)),
        ],
        out_specs=[
            pl.BlockSpec((1, OUT, BL), lambda f, j: (f, 0, j)),
            pl.BlockSpec((1, HEADS, BL), lambda f, j: (f, 0, j)),
            pl.BlockSpec((1, HEADS, BL), lambda f, j: (f, 0, j)),
        ],
        out_shape=[
            jax.ShapeDtypeStruct((FRAMES, OUT, NP), f32),
            jax.ShapeDtypeStruct((FRAMES, HEADS, NP), f32),
            jax.ShapeDtypeStruct((FRAMES, HEADS, NP), f32),
        ],
    )(xp, W1T, AsT, AdT)


def _tc2_body(o_ref, b1_ref, w2_ref, as2_ref, ad2_ref, h2_ref, als_ref, ald_ref):
    h = o_ref[...] + b1_ref[...]
    h = jnp.where(h > 0, h, jnp.exp(h) - 1.0)
    h2 = jnp.dot(w2_ref[...], h, preferred_element_type=f32)
    h2_ref[...] = h2
    als_ref[...] = jnp.dot(as2_ref[...], h2, preferred_element_type=f32)
    ald_ref[...] = jnp.dot(ad2_ref[...], h2, preferred_element_type=f32)


def _tc2(out1T, b1c, W2T, as2, ad2):
    return pl.pallas_call(
        _tc2_body,
        grid=(NP // BL,),
        in_specs=[
            pl.BlockSpec((OUT, BL), lambda j: (0, j)),
            pl.BlockSpec((OUT, 1), lambda j: (0, 0)),
            pl.BlockSpec((OUT, OUT), lambda j: (0, 0)),
            pl.BlockSpec((1, OUT), lambda j: (0, 0)),
            pl.BlockSpec((1, OUT), lambda j: (0, 0)),
        ],
        out_specs=[
            pl.BlockSpec((OUT, BL), lambda j: (0, j)),
            pl.BlockSpec((1, BL), lambda j: (0, j)),
            pl.BlockSpec((1, BL), lambda j: (0, j)),
        ],
        out_shape=[
            jax.ShapeDtypeStruct((OUT, NP), f32),
            jax.ShapeDtypeStruct((1, NP), f32),
            jax.ShapeDtypeStruct((1, NP), f32),
        ],
    )(out1T, b1c, W2T, as2, ad2)


def _ln_in(h, g, b):
    m = jnp.mean(h, axis=1, keepdims=True)
    v = jnp.mean((h - m) ** 2, axis=1, keepdims=True)
    return (h - m) * lax.rsqrt(v + 1e-5) * g + b


def _tc3_body(h_ref, Wq_ref, bq_ref, Wk_ref, bk_ref, Wv_ref, bv_ref,
              Wo_ref, bo_ref, g1_ref, be1_ref, g2_ref, be2_ref,
              Wf1_ref, bf1_ref, Wf2_ref, bf2_ref,
              Wh1_ref, bh1_ref, Wh2_ref, bh2_ref, out_ref):
    h = h_ref[...]
    hd = OUT // NHEAD
    for layer in range(NLAYERS):
        q = jnp.dot(h, Wq_ref[layer], preferred_element_type=f32) + bq_ref[layer]
        k = jnp.dot(h, Wk_ref[layer], preferred_element_type=f32) + bk_ref[layer]
        v = jnp.dot(h, Wv_ref[layer], preferred_element_type=f32) + bv_ref[layer]
        outs = []
        for hh in range(NHEAD):
            sl = slice(hh * hd, (hh + 1) * hd)
            qh, kh, vh = q[:, sl], k[:, sl], v[:, sl]
            att = lax.dot_general(qh, kh, (((1,), (1,)), ((), ())),
                                  preferred_element_type=f32) * (1.0 / (hd ** 0.5))
            att = att - jnp.max(att, axis=1, keepdims=True)
            att = jnp.exp(att)
            att = att / jnp.sum(att, axis=1, keepdims=True)
            outs.append(lax.dot_general(att, vh, (((1,), (0,)), ((), ())),
                                        preferred_element_type=f32))
        o = jnp.concatenate(outs, axis=1)
        o = jnp.dot(o, Wo_ref[layer], preferred_element_type=f32) + bo_ref[layer]
        h = _ln_in(h + o, g1_ref[layer], be1_ref[layer])
        f1 = jnp.maximum(jnp.dot(h, Wf1_ref[layer], preferred_element_type=f32)
                         + bf1_ref[layer], 0.0)
        fo = jnp.dot(f1, Wf2_ref[layer], preferred_element_type=f32) + bf2_ref[layer]
        h = _ln_in(h + fo, g2_ref[layer], be2_ref[layer])
    last = h[FRAMES - 1:FRAMES, :]
    r = jnp.maximum(jnp.dot(last, Wh1_ref[...], preferred_element_type=f32)
                    + bh1_ref[...], 0.0)
    out_ref[...] = jnp.dot(r, Wh2_ref[...], preferred_element_type=f32) + bh2_ref[...]


def _tc3(hseq, Wq, bq, Wk, bk, Wv, bv, Wo, bo, g1, be1, g2, be2,
         Wf1, bf1, Wf2, bf2, Wh1, bh1, Wh2, bh2):
    return pl.pallas_call(
        _tc3_body,
        out_shape=jax.ShapeDtypeStruct((1, 2), f32),
    )(hseq, Wq, bq, Wk, bk, Wv, bv, Wo, bo, g1, be1, g2, be2,
      Wf1, bf1, Wf2, bf2, Wh1, bh1, Wh2, bh2)


_MESH = plsc.VectorSubcoreMesh(core_axis_name="c", subcore_axis_name="s")


def _zero_range(ref, n):
    z = jnp.zeros((L,), f32)

    @pl.loop(0, n // L)
    def _(i):
        ref[pl.ds(i * L, L)] = z


def _edge_stream(src_h, dst_h, srcb, dstb, sems, semd, base, nch, process):
    pltpu.async_copy(src_h.at[pl.ds(base, C)], srcb.at[0], sems.at[0])
    pltpu.async_copy(dst_h.at[pl.ds(base, C)], dstb.at[0], semd.at[0])

    @pl.loop(0, nch, step=2)
    def _(ci):
        off0 = base + ci * C
        pltpu.make_async_copy(src_h.at[pl.ds(off0, C)], srcb.at[0], sems.at[0]).wait()
        pltpu.make_async_copy(dst_h.at[pl.ds(off0, C)], dstb.at[0], semd.at[0]).wait()

        @pl.when(ci + 1 < nch)
        def _():
            pltpu.async_copy(src_h.at[pl.ds(off0 + C, C)], srcb.at[1], sems.at[1])
            pltpu.async_copy(dst_h.at[pl.ds(off0 + C, C)], dstb.at[1], semd.at[1])

        process(off0, 0)

        @pl.when(ci + 1 < nch)
        def _():
            pltpu.make_async_copy(src_h.at[pl.ds(off0 + C, C)], srcb.at[1], sems.at[1]).wait()
            pltpu.make_async_copy(dst_h.at[pl.ds(off0 + C, C)], dstb.at[1], semd.at[1]).wait()

            @pl.when(ci + 2 < nch)
            def _():
                pltpu.async_copy(src_h.at[pl.ds(off0 + 2 * C, C)], srcb.at[0], sems.at[0])
                pltpu.async_copy(dst_h.at[pl.ds(off0 + 2 * C, C)], dstb.at[0], semd.at[0])

            process(off0 + C, 1)


_SC_SCRATCH = [
    pltpu.VMEM((NP,), f32),
    pltpu.VMEM((NP,), f32),
    pltpu.VMEM((NP,), f32),
    pltpu.VMEM((NP,), f32),
    pltpu.VMEM((NP,), f32),
    pltpu.VMEM((NP,), f32),
    pltpu.VMEM((NP,), f32),
    pltpu.VMEM((NP // 4,), f32),
    pltpu.VMEM((2, C), jnp.int32),
    pltpu.VMEM((2, C), jnp.int32),
    pltpu.VMEM_SHARED((16, NP), f32),
    pltpu.VMEM_SHARED((4, NP), f32),
    pltpu.SemaphoreType.DMA((2,)),
    pltpu.SemaphoreType.DMA((2,)),
]


@functools.partial(
    pl.kernel,
    out_type=jax.ShapeDtypeStruct((OUT, NP), f32),
    mesh=_MESH,
    scratch_types=_SC_SCRATCH,
    compiler_params=pltpu.CompilerParams(needs_layout_passes=False),
)
def _sc_l1(src_h, dst_h, alsT_h, aldT_h, h1T_h, out_h,
           als_t, ald_t, den_t, hc0, hc1, o0, o1, tmpc, srcb, dstb,
           shpart, shcomb, sems, semd):
    c = lax.axis_index("c")
    s = lax.axis_index("s")
    hloc = s // 4
    ghead = 4 * c + hloc
    col0 = 32 * c + 2 * s

    pltpu.sync_copy(alsT_h.at[ghead], als_t)
    pltpu.sync_copy(aldT_h.at[ghead], ald_t)
    _zero_range(den_t, NP)
    _zero_range(o0, NP)
    _zero_range(o1, NP)

    qa = s % 4
    EQ = E_PAD // 4

    def _procA(off, b):
        for j in range(C // L):
            s16 = srcb[b, pl.ds(j * L, L)]
            d16 = dstb[b, pl.ds(j * L, L)]
            a = plsc.load_gather(als_t, [s16]) + plsc.load_gather(ald_t, [d16])
            a = jnp.where(a > 0, a, 0.2 * a)
            ex = jnp.exp(a)
            eid = off + j * L + lax.iota(jnp.int32, L)
            plsc.addupdate_scatter(den_t, [d16], ex, mask=eid < E_TOT)

    _edge_stream(src_h, dst_h, srcb, dstb, sems, semd, qa * EQ, EQ // C, _procA)

    pltpu.sync_copy(den_t, shpart.at[s])
    plsc.subcore_barrier()
    SL1 = NP // 4
    offn = qa * SL1
    pltpu.sync_copy(shpart.at[4 * hloc, pl.ds(offn, SL1)], hc0.at[pl.ds(0, SL1)])
    for k2 in range(1, 4):
        pltpu.sync_copy(shpart.at[4 * hloc + k2, pl.ds(offn, SL1)],
                        tmpc.at[pl.ds(0, SL1)])

        @pl.loop(0, SL1 // L)
        def _(i):
            hc0[pl.ds(i * L, L)] = hc0[pl.ds(i * L, L)] + tmpc[pl.ds(i * L, L)]

    pltpu.sync_copy(hc0.at[pl.ds(0, SL1)], shcomb.at[hloc, pl.ds(offn, SL1)])
    plsc.subcore_barrier()
    pltpu.sync_copy(shcomb.at[hloc], den_t)

    pltpu.sync_copy(h1T_h.at[col0], hc0)
    pltpu.sync_copy(h1T_h.at[col0 + 1], hc1)

    def _procB(off, b):
        for j in range(C // L):
            s16 = srcb[b, pl.ds(j * L, L)]
            d16 = dstb[b, pl.ds(j * L, L)]
            a = plsc.load_gather(als_t, [s16]) + plsc.load_gather(ald_t, [d16])
            a = jnp.where(a > 0, a, 0.2 * a)
            ex = jnp.exp(a)
            dn = plsc.load_gather(den_t, [d16])
            coef = ex / (dn + 1e-16)
            eid = off + j * L + lax.iota(jnp.int32, L)
            valid = eid < E_TOT
            g0 = plsc.load_gather(hc0, [s16])
            g1 = plsc.load_gather(hc1, [s16])
            plsc.addupdate_scatter(o0, [d16], coef * g0, mask=valid)
            plsc.addupdate_scatter(o1, [d16], coef * g1, mask=valid)

    _edge_stream(src_h, dst_h, srcb, dstb, sems, semd, 0, E_PAD // C, _procB)

    pltpu.sync_copy(o0, out_h.at[col0])
    pltpu.sync_copy(o1, out_h.at[col0 + 1])


@functools.partial(
    pl.kernel,
    out_type=jax.ShapeDtypeStruct((OUT * L,), f32),
    mesh=_MESH,
    scratch_types=_SC_SCRATCH,
    compiler_params=pltpu.CompilerParams(needs_layout_passes=False),
)
def _sc_l2(src_h, dst_h, als2_h, ald2_h, h2T_h, out_h,
           als_t, ald_t, den_t, hc0, hc1, o0, o1, tmpc, srcb, dstb,
           shpart, shcomb, sems, semd):
    c = lax.axis_index("c")
    s = lax.axis_index("s")
    col0 = 32 * c + 2 * s

    pltpu.sync_copy(als2_h, als_t)
    pltpu.sync_copy(ald2_h, ald_t)
    _zero_range(den_t, NP)

    ES = E_PAD // 16

    def _procA(off, b):
        for j in range(C // L):
            s16 = srcb[b, pl.ds(j * L, L)]
            d16 = dstb[b, pl.ds(j * L, L)]
            a = plsc.load_gather(als_t, [s16]) + plsc.load_gather(ald_t, [d16])
            a = jnp.where(a > 0, a, 0.2 * a)
            ex = jnp.exp(a)
            eid = off + j * L + lax.iota(jnp.int32, L)
            plsc.addupdate_scatter(den_t, [d16], ex, mask=eid < E_TOT)

    _edge_stream(src_h, dst_h, srcb, dstb, sems, semd, s * ES, ES // C, _procA)

    pltpu.sync_copy(den_t, shpart.at[s])
    plsc.subcore_barrier()
    SL2 = NP // 16
    offn = s * SL2
    pltpu.sync_copy(shpart.at[0, pl.ds(offn, SL2)], hc0.at[pl.ds(0, SL2)])
    for k2 in range(1, 16):
        pltpu.sync_copy(shpart.at[k2, pl.ds(offn, SL2)], tmpc.at[pl.ds(0, SL2)])

        @pl.loop(0, SL2 // L)
        def _(i):
            hc0[pl.ds(i * L, L)] = hc0[pl.ds(i * L, L)] + tmpc[pl.ds(i * L, L)]

    pltpu.sync_copy(hc0.at[pl.ds(0, SL2)], shcomb.at[0, pl.ds(offn, SL2)])
    plsc.subcore_barrier()
    pltpu.sync_copy(shcomb.at[0], den_t)

    pltpu.sync_copy(h2T_h.at[col0], hc0)
    pltpu.sync_copy(h2T_h.at[col0 + 1], hc1)

    def _procB(off, b):
        acc0 = o0[pl.ds(0, L)]
        acc1 = o1[pl.ds(0, L)]
        for j in range(C // L):
            s16 = srcb[b, pl.ds(j * L, L)]
            d16 = dstb[b, pl.ds(j * L, L)]
            a = plsc.load_gather(als_t, [s16]) + plsc.load_gather(ald_t, [d16])
            a = jnp.where(a > 0, a, 0.2 * a)
            ex = jnp.exp(a)
            dn = plsc.load_gather(den_t, [d16])
            eid = off + j * L + lax.iota(jnp.int32, L)
            coef = jnp.where(eid < E_TOT, ex / (dn + 1e-16), 0.0)
            acc0 = acc0 + coef * plsc.load_gather(hc0, [s16])
            acc1 = acc1 + coef * plsc.load_gather(hc1, [s16])
        o0[pl.ds(0, L)] = acc0
        o1[pl.ds(0, L)] = acc1

    z = jnp.zeros((L,), f32)
    o0[pl.ds(0, L)] = z
    o1[pl.ds(0, L)] = z
    _edge_stream(src_h, dst_h, srcb, dstb, sems, semd, 0, E_PAD // C, _procB)

    pltpu.sync_copy(o0.at[pl.ds(0, L)], out_h.at[pl.ds(col0 * L, L)])
    pltpu.sync_copy(o1.at[pl.ds(0, L)], out_h.at[pl.ds((col0 + 1) * L, L)])


def kernel(x, edge_index, W1, att_src1, att_dst1, b1, W2, att_src2, att_dst2,
           b2, Wq, bq, Wk, bk, Wv, bv, Wo, bo, ln1_g, ln1_b, ln2_g, ln2_b,
           Wff1, bff1, Wff2, bff2, Whead1, bhead1, Whead2, bhead2):
    xp = jnp.pad(x, ((0, 0), (0, NP - N), (0, 0)))
    W1T = W1.T
    eye8 = jnp.eye(HEADS, dtype=f32)
    AsT = (eye8[:, :, None] * att_src1[None, :, :]).reshape(HEADS, OUT)
    AdT = (eye8[:, :, None] * att_dst1[None, :, :]).reshape(HEADS, OUT)
    h1T, alsT, aldT = _tc1(xp, W1T, AsT, AdT)

    loop = jnp.arange(N, dtype=jnp.int32)
    loops = jnp.broadcast_to(loop, (FRAMES, N))
    srcs = jnp.pad(jnp.concatenate([edge_index[:, 0, :], loops], axis=1),
                   ((0, 0), (0, E_PAD - E_TOT)))
    dsts = jnp.pad(jnp.concatenate([edge_index[:, 1, :], loops], axis=1),
                   ((0, 0), (0, E_PAD - E_TOT)))

    W2T = W2.T
    b1c = b1.reshape(OUT, 1)
    embs = []
    for f in range(FRAMES):
        out1T = _sc_l1(srcs[f], dsts[f], alsT[f], aldT[f], h1T[f])
        h2T, als2, ald2 = _tc2(out1T, b1c, W2T, att_src2, att_dst2)
        sc2 = _sc_l2(srcs[f], dsts[f], als2.reshape(NP), ald2.reshape(NP), h2T)
        embs.append(sc2.reshape(OUT, L).sum(axis=1) * (1.0 / N) + b2)
    hseq = jnp.stack(embs)

    return _tc3(hseq, Wq, bq, Wk, bk, Wv, bv, Wo, bo,
                ln1_g, ln1_b, ln2_g, ln2_b, Wff1, bff1, Wff2, bff2,
                Whead1, bhead1, Whead2, bhead2)


# speedup vs baseline: 53.5155x; 1.4415x over previous
import functools

import jax
import jax.numpy as jnp
from jax import lax
from jax.experimental import pallas as pl
from jax.experimental.pallas import tpu as pltpu
from jax.experimental.pallas import tpu_sc as plsc

N = 10000
NP = 10240
E_RAW = 320000
E_TOT = E_RAW + N
C = 1024
NCH = 336
E_PAD = NCH * C
NB = 7
HEADS, HID, OUT = 8, 8, 64
NHEAD, NLAYERS, DFF = 4, 3, 128
FRAMES = 5
L = 16
JN = C // L
BL = 2048

f32 = jnp.float32


def _tc1_body(x_ref, w_ref, as_ref, ad_ref, h_ref, als_ref, ald_ref):
    xb = x_ref[0]
    h = lax.dot_general(w_ref[...], xb, (((1,), (1,)), ((), ())),
                        preferred_element_type=f32)
    h_ref[0] = h
    als_ref[0] = jnp.dot(as_ref[...], h, preferred_element_type=f32)
    ald_ref[0] = jnp.dot(ad_ref[...], h, preferred_element_type=f32)


def _tc1(xp, W1T, AsT, AdT):
    grid = (FRAMES, NP // BL)
    return pl.pallas_call(
        _tc1_body,
        grid=grid,
        in_specs=[
            pl.BlockSpec((1, BL, 128), lambda f, j: (f, j, 0)),
            pl.BlockSpec((OUT, 128), lambda f, j: (0, 0)),
            pl.BlockSpec((HEADS, OUT), lambda f, j: (0, 0)),
            pl.BlockSpec((HEADS, OUT), lambda f, j: (0, 0)),
        ],
        out_specs=[
            pl.BlockSpec((1, OUT, BL), lambda f, j: (f, 0, j)),
            pl.BlockSpec((1, HEADS, BL), lambda f, j: (f, 0, j)),
            pl.BlockSpec((1, HEADS, BL), lambda f, j: (f, 0, j)),
        ],
        out_shape=[
            jax.ShapeDtypeStruct((FRAMES, OUT, NP), f32),
            jax.ShapeDtypeStruct((FRAMES, HEADS, NP), f32),
            jax.ShapeDtypeStruct((FRAMES, HEADS, NP), f32),
        ],
    )(xp, W1T, AsT, AdT)


def _tc2_body(o_ref, b1_ref, w2_ref, as2_ref, ad2_ref, h2_ref, als_ref, ald_ref):
    h = o_ref[...] + b1_ref[...]
    h = jnp.where(h > 0, h, jnp.exp(h) - 1.0)
    h2 = jnp.dot(w2_ref[...], h, preferred_element_type=f32)
    h2_ref[...] = h2
    als_ref[...] = jnp.dot(as2_ref[...], h2, preferred_element_type=f32)
    ald_ref[...] = jnp.dot(ad2_ref[...], h2, preferred_element_type=f32)


def _tc2(out1T, b1c, W2T, as2, ad2):
    return pl.pallas_call(
        _tc2_body,
        grid=(NP // BL,),
        in_specs=[
            pl.BlockSpec((OUT, BL), lambda j: (0, j)),
            pl.BlockSpec((OUT, 1), lambda j: (0, 0)),
            pl.BlockSpec((OUT, OUT), lambda j: (0, 0)),
            pl.BlockSpec((1, OUT), lambda j: (0, 0)),
            pl.BlockSpec((1, OUT), lambda j: (0, 0)),
        ],
        out_specs=[
            pl.BlockSpec((OUT, BL), lambda j: (0, j)),
            pl.BlockSpec((1, BL), lambda j: (0, j)),
            pl.BlockSpec((1, BL), lambda j: (0, j)),
        ],
        out_shape=[
            jax.ShapeDtypeStruct((OUT, NP), f32),
            jax.ShapeDtypeStruct((1, NP), f32),
            jax.ShapeDtypeStruct((1, NP), f32),
        ],
    )(out1T, b1c, W2T, as2, ad2)


def _ln_in(h, g, b):
    m = jnp.mean(h, axis=1, keepdims=True)
    v = jnp.mean((h - m) ** 2, axis=1, keepdims=True)
    return (h - m) * lax.rsqrt(v + 1e-5) * g + b


def _tc3_body(h_ref, Wq_ref, bq_ref, Wk_ref, bk_ref, Wv_ref, bv_ref,
              Wo_ref, bo_ref, g1_ref, be1_ref, g2_ref, be2_ref,
              Wf1_ref, bf1_ref, Wf2_ref, bf2_ref,
              Wh1_ref, bh1_ref, Wh2_ref, bh2_ref, out_ref):
    h = h_ref[...]
    hd = OUT // NHEAD
    for layer in range(NLAYERS):
        q = jnp.dot(h, Wq_ref[layer], preferred_element_type=f32) + bq_ref[layer]
        k = jnp.dot(h, Wk_ref[layer], preferred_element_type=f32) + bk_ref[layer]
        v = jnp.dot(h, Wv_ref[layer], preferred_element_type=f32) + bv_ref[layer]
        outs = []
        for hh in range(NHEAD):
            sl = slice(hh * hd, (hh + 1) * hd)
            qh, kh, vh = q[:, sl], k[:, sl], v[:, sl]
            att = lax.dot_general(qh, kh, (((1,), (1,)), ((), ())),
                                  preferred_element_type=f32) * (1.0 / (hd ** 0.5))
            att = att - jnp.max(att, axis=1, keepdims=True)
            att = jnp.exp(att)
            att = att / jnp.sum(att, axis=1, keepdims=True)
            outs.append(lax.dot_general(att, vh, (((1,), (0,)), ((), ())),
                                        preferred_element_type=f32))
        o = jnp.concatenate(outs, axis=1)
        o = jnp.dot(o, Wo_ref[layer], preferred_element_type=f32) + bo_ref[layer]
        h = _ln_in(h + o, g1_ref[layer], be1_ref[layer])
        f1 = jnp.maximum(jnp.dot(h, Wf1_ref[layer], preferred_element_type=f32)
                         + bf1_ref[layer], 0.0)
        fo = jnp.dot(f1, Wf2_ref[layer], preferred_element_type=f32) + bf2_ref[layer]
        h = _ln_in(h + fo, g2_ref[layer], be2_ref[layer])
    last = h[FRAMES - 1:FRAMES, :]
    r = jnp.maximum(jnp.dot(last, Wh1_ref[...], preferred_element_type=f32)
                    + bh1_ref[...], 0.0)
    out_ref[...] = jnp.dot(r, Wh2_ref[...], preferred_element_type=f32) + bh2_ref[...]


def _tc3(hseq, Wq, bq, Wk, bk, Wv, bv, Wo, bo, g1, be1, g2, be2,
         Wf1, bf1, Wf2, bf2, Wh1, bh1, Wh2, bh2):
    return pl.pallas_call(
        _tc3_body,
        out_shape=jax.ShapeDtypeStruct((1, 2), f32),
    )(hseq, Wq, bq, Wk, bk, Wv, bv, Wo, bo, g1, be1, g2, be2,
      Wf1, bf1, Wf2, bf2, Wh1, bh1, Wh2, bh2)


_MESH = plsc.VectorSubcoreMesh(core_axis_name="c", subcore_axis_name="s")


def _zero_range(ref, lo, n):
    z = jnp.zeros((L,), f32)

    @pl.loop(0, n // L)
    def _(i):
        ref[pl.ds(lo + i * L, L)] = z


def _edge_stream(pk_h, bufs, sems, base, nch, process):
    for k in range(NB):
        pltpu.async_copy(pk_h.at[base + k], bufs.at[pl.ds(k * 2 * C, 2 * C)], sems.at[k])

    @pl.loop(0, nch, step=NB)
    def _(ci):
        for k in range(NB):
            ch = base + ci + k
            pltpu.make_async_copy(pk_h.at[ch], bufs.at[pl.ds(k * 2 * C, 2 * C)], sems.at[k]).wait()
            process(ci + k, k)

            @pl.when(ci + k + NB < nch)
            def _():
                pltpu.async_copy(pk_h.at[ch + NB], bufs.at[pl.ds(k * 2 * C, 2 * C)], sems.at[k])


_SC_SCRATCH = [
    pltpu.VMEM((NP,), f32),
    pltpu.VMEM((NP,), f32),
    pltpu.VMEM((NP,), f32),
    pltpu.VMEM((NP,), f32),
    pltpu.VMEM((NP,), f32),
    pltpu.VMEM((NP,), f32),
    pltpu.VMEM((NP,), f32),
    pltpu.VMEM((NP // 4,), f32),
    pltpu.VMEM((NB * 2 * C,), jnp.int32),
    pltpu.VMEM_SHARED((16, NP), f32),
    pltpu.VMEM_SHARED((4, NP), f32),
    pltpu.SemaphoreType.DMA((NB,)),
]


@functools.partial(
    pl.kernel,
    out_type=jax.ShapeDtypeStruct((OUT, NP), f32),
    mesh=_MESH,
    scratch_types=_SC_SCRATCH,
    compiler_params=pltpu.CompilerParams(needs_layout_passes=False),
)
def _sc_l1(pk_h, alsT_h, aldT_h, h1T_h, out_h,
           als_t, ald_t, den_t, hc0, hc1, o0, o1, tmpc, bufs,
           shpart, shcomb, sems):
    c = lax.axis_index("c")
    s = lax.axis_index("s")
    hloc = s // 4
    ghead = 4 * c + hloc
    col0 = 32 * c + 2 * s

    pltpu.sync_copy(alsT_h.at[ghead], als_t)
    pltpu.sync_copy(aldT_h.at[ghead], ald_t)
    _zero_range(den_t, 0, NP)
    _zero_range(o0, 0, NP)
    _zero_range(o1, 0, NP)

    qa = s % 4
    NQ = NCH // 4

    def _procA(ch, k):
        @pl.loop(0, JN, unroll=8)
        def _(j):
            s16 = bufs[pl.ds(k * 2 * C + j * L, L)]
            d16 = bufs[pl.ds(k * 2 * C + C + j * L, L)]
            a = plsc.load_gather(als_t, [s16]) + plsc.load_gather(ald_t, [d16])
            a = jnp.maximum(a, 0.2 * a)
            plsc.addupdate_scatter(den_t, [d16], jnp.exp(a))

    _edge_stream(pk_h, bufs, sems, qa * NQ, NQ, _procA)

    pltpu.sync_copy(den_t, shpart.at[s])
    plsc.subcore_barrier()
    SL1 = NP // 4
    offn = qa * SL1
    pltpu.sync_copy(shpart.at[4 * hloc, pl.ds(offn, SL1)], hc0.at[pl.ds(0, SL1)])
    for k2 in range(1, 4):
        pltpu.sync_copy(shpart.at[4 * hloc + k2, pl.ds(offn, SL1)],
                        tmpc.at[pl.ds(0, SL1)])

        @pl.loop(0, SL1 // L)
        def _(i):
            hc0[pl.ds(i * L, L)] = hc0[pl.ds(i * L, L)] + tmpc[pl.ds(i * L, L)]

    pltpu.sync_copy(hc0.at[pl.ds(0, SL1)], shcomb.at[hloc, pl.ds(offn, SL1)])
    plsc.subcore_barrier()
    pltpu.sync_copy(shcomb.at[hloc], den_t)

    pltpu.sync_copy(h1T_h.at[col0], hc0)
    pltpu.sync_copy(h1T_h.at[col0 + 1], hc1)

    def _procB(ch, k):
        @pl.loop(0, JN, unroll=8)
        def _(j):
            s16 = bufs[pl.ds(k * 2 * C + j * L, L)]
            d16 = bufs[pl.ds(k * 2 * C + C + j * L, L)]
            a = plsc.load_gather(als_t, [s16]) + plsc.load_gather(ald_t, [d16])
            a = jnp.maximum(a, 0.2 * a)
            ex = jnp.exp(a)
            dn = plsc.load_gather(den_t, [d16])
            coef = ex / (dn + 1e-16)
            g0 = plsc.load_gather(hc0, [s16])
            g1 = plsc.load_gather(hc1, [s16])
            plsc.addupdate_scatter(o0, [d16], coef * g0)
            plsc.addupdate_scatter(o1, [d16], coef * g1)

    _edge_stream(pk_h, bufs, sems, 0, NCH, _procB)

    pltpu.sync_copy(o0, out_h.at[col0])
    pltpu.sync_copy(o1, out_h.at[col0 + 1])


@functools.partial(
    pl.kernel,
    out_type=jax.ShapeDtypeStruct((OUT * L,), f32),
    mesh=_MESH,
    scratch_types=_SC_SCRATCH,
    compiler_params=pltpu.CompilerParams(needs_layout_passes=False),
)
def _sc_l2(pk_h, als2_h, ald2_h, h2T_h, out_h,
           als_t, ald_t, den_t, hc0, hc1, o0, o1, tmpc, bufs,
           shpart, shcomb, sems):
    c = lax.axis_index("c")
    s = lax.axis_index("s")
    col0 = 32 * c + 2 * s

    pltpu.sync_copy(als2_h, als_t)
    pltpu.sync_copy(ald2_h, ald_t)
    _zero_range(den_t, 0, NP)

    NS16 = NCH // 16

    def _procA(ch, k):
        @pl.loop(0, JN, unroll=8)
        def _(j):
            s16 = bufs[pl.ds(k * 2 * C + j * L, L)]
            d16 = bufs[pl.ds(k * 2 * C + C + j * L, L)]
            a = plsc.load_gather(als_t, [s16]) + plsc.load_gather(ald_t, [d16])
            a = jnp.maximum(a, 0.2 * a)
            plsc.addupdate_scatter(den_t, [d16], jnp.exp(a))

    _edge_stream(pk_h, bufs, sems, s * NS16, NS16, _procA)

    pltpu.sync_copy(den_t, shpart.at[s])
    plsc.subcore_barrier()
    SL2 = NP // 16
    offn = s * SL2
    pltpu.sync_copy(shpart.at[0, pl.ds(offn, SL2)], hc0.at[pl.ds(0, SL2)])
    for k2 in range(1, 16):
        pltpu.sync_copy(shpart.at[k2, pl.ds(offn, SL2)], tmpc.at[pl.ds(0, SL2)])

        @pl.loop(0, SL2 // L)
        def _(i):
            hc0[pl.ds(i * L, L)] = hc0[pl.ds(i * L, L)] + tmpc[pl.ds(i * L, L)]

    pltpu.sync_copy(hc0.at[pl.ds(0, SL2)], shcomb.at[0, pl.ds(offn, SL2)])
    plsc.subcore_barrier()
    pltpu.sync_copy(shcomb.at[0], den_t)

    pltpu.sync_copy(h2T_h.at[col0], hc0)
    pltpu.sync_copy(h2T_h.at[col0 + 1], hc1)
    _zero_range(hc0, N, NP - N)
    _zero_range(hc1, N, NP - N)

    def _procB(ch, k):
        def jbody(j, carry):
            a0, a1 = carry
            s16 = bufs[pl.ds(k * 2 * C + j * L, L)]
            d16 = bufs[pl.ds(k * 2 * C + C + j * L, L)]
            a = plsc.load_gather(als_t, [s16]) + plsc.load_gather(ald_t, [d16])
            a = jnp.maximum(a, 0.2 * a)
            ex = jnp.exp(a)
            dn = plsc.load_gather(den_t, [d16])
            coef = ex / (dn + 1e-16)
            a0 = a0 + coef * plsc.load_gather(hc0, [s16])
            a1 = a1 + coef * plsc.load_gather(hc1, [s16])
            return (a0, a1)

        acc = pl.loop(0, JN, init_carry=(o0[pl.ds(0, L)], o1[pl.ds(0, L)]),
                      unroll=8)(jbody)
        o0[pl.ds(0, L)] = acc[0]
        o1[pl.ds(0, L)] = acc[1]

    z = jnp.zeros((L,), f32)
    o0[pl.ds(0, L)] = z
    o1[pl.ds(0, L)] = z
    _edge_stream(pk_h, bufs, sems, 0, NCH, _procB)

    pltpu.sync_copy(o0.at[pl.ds(0, L)], out_h.at[pl.ds(col0 * L, L)])
    pltpu.sync_copy(o1.at[pl.ds(0, L)], out_h.at[pl.ds((col0 + 1) * L, L)])


def kernel(x, edge_index, W1, att_src1, att_dst1, b1, W2, att_src2, att_dst2,
           b2, Wq, bq, Wk, bk, Wv, bv, Wo, bo, ln1_g, ln1_b, ln2_g, ln2_b,
           Wff1, bff1, Wff2, bff2, Whead1, bhead1, Whead2, bhead2):
    xp = jnp.pad(x, ((0, 0), (0, NP - N), (0, 0)))
    W1T = W1.T
    eye8 = jnp.eye(HEADS, dtype=f32)
    AsT = (eye8[:, :, None] * att_src1[None, :, :]).reshape(HEADS, OUT)
    AdT = (eye8[:, :, None] * att_dst1[None, :, :]).reshape(HEADS, OUT)
    h1T, alsT, aldT = _tc1(xp, W1T, AsT, AdT)

    loop = jnp.arange(N, dtype=jnp.int32)
    loops = jnp.broadcast_to(loop, (FRAMES, N))
    srcs = jnp.pad(jnp.concatenate([edge_index[:, 0, :], loops], axis=1),
                   ((0, 0), (0, E_PAD - E_TOT)), constant_values=N)
    dsts = jnp.pad(jnp.concatenate([edge_index[:, 1, :], loops], axis=1),
                   ((0, 0), (0, E_PAD - E_TOT)), constant_values=N)
    packed = jnp.stack([srcs.reshape(FRAMES, NCH, C),
                        dsts.reshape(FRAMES, NCH, C)], axis=2)
    packed = packed.reshape(FRAMES, NCH, 2 * C)

    W2T = W2.T
    b1c = b1.reshape(OUT, 1)
    embs = []
    for f in range(FRAMES):
        out1T = _sc_l1(packed[f], alsT[f], aldT[f], h1T[f])
        h2T, als2, ald2 = _tc2(out1T, b1c, W2T, att_src2, att_dst2)
        sc2 = _sc_l2(packed[f], als2.reshape(NP), ald2.reshape(NP), h2T)
        embs.append(sc2.reshape(OUT, L).sum(axis=1) * (1.0 / N) + b2)
    hseq = jnp.stack(embs)

    return _tc3(hseq, Wq, bq, Wk, bk, Wv, bv, Wo, bo,
                ln1_g, ln1_b, ln2_g, ln2_b, Wff1, bff1, Wff2, bff2,
                Whead1, bhead1, Whead2, bhead2)
